# Initial kernel scaffold; baseline (speedup 1.0000x reference)
import jax
import jax.numpy as jnp
from jax.experimental import pallas as pl


def kernel(radius, table):
    raise NotImplementedError("write your pallas kernel here")


# speedup vs baseline: 6.3287x; 6.3287x over previous
import functools

import jax
import jax.numpy as jnp
from jax import lax
from jax.experimental import pallas as pl
from jax.experimental.pallas import tpu as pltpu
from jax.experimental.pallas import tpu_sc as plsc

EMBED_DIM = 32
CHUNK = 2048


@functools.cache
def _build(B: int, V: int, D: int):
    info = plsc.get_sparse_core_info()
    NW = info.num_cores * info.num_subcores
    b_per_w = B // NW
    n_chunks = b_per_w // CHUNK
    mesh = plsc.VectorSubcoreMesh(core_axis_name="c", subcore_axis_name="s")

    @functools.partial(
        pl.kernel,
        mesh=mesh,
        out_type=jax.ShapeDtypeStruct((B, D), jnp.float32),
        scratch_types=[
            pltpu.VMEM((CHUNK,), jnp.int32),
            pltpu.VMEM((CHUNK, D), jnp.float32),
            pltpu.SemaphoreType.DMA,
        ],
        compiler_params=pltpu.CompilerParams(use_tc_tiling_on_sc=False),
    )
    def gather_kernel(idx_hbm, table_hbm, out_hbm, idx_v, rows_v, sem):
        wid = lax.axis_index("s") * info.num_cores + lax.axis_index("c")
        base = wid * b_per_w

        def body(i, carry):
            off = base + i * CHUNK
            pltpu.sync_copy(idx_hbm.at[pl.ds(off, CHUNK)], idx_v)
            pltpu.async_copy(table_hbm.at[idx_v], rows_v, sem).wait()
            pltpu.sync_copy(rows_v, out_hbm.at[pl.ds(off, CHUNK)])
            return carry

        lax.fori_loop(0, n_chunks, body, 0)

    return gather_kernel


def kernel(radius, table):
    B0, H = radius.shape
    V, D = table.shape
    flat_idx = radius.reshape(-1).astype(jnp.int32)
    out = _build(B0 * H, V, D)(flat_idx, table)
    return out.reshape(B0, H, D)


# speedup vs baseline: 6.4893x; 1.0254x over previous
import functools

import jax
import jax.numpy as jnp
from jax import lax
from jax.experimental import pallas as pl
from jax.experimental.pallas import tpu as pltpu
from jax.experimental.pallas import tpu_sc as plsc

CHUNK = 1600


@functools.cache
def _build(B: int, V: int, D: int):
    info = plsc.get_sparse_core_info()
    NW = info.num_cores * info.num_subcores
    b_per_w = B // NW
    n_chunks = b_per_w // CHUNK
    assert n_chunks % 2 == 0
    pairs = n_chunks // 2
    mesh = plsc.VectorSubcoreMesh(core_axis_name="c", subcore_axis_name="s")

    @functools.partial(
        pl.kernel,
        mesh=mesh,
        out_type=jax.ShapeDtypeStruct((B, D), jnp.float32),
        scratch_types=[
            pltpu.VMEM((CHUNK,), jnp.int32),
            pltpu.VMEM((CHUNK,), jnp.int32),
            pltpu.VMEM((CHUNK, D), jnp.float32),
            pltpu.VMEM((CHUNK, D), jnp.float32),
            pltpu.SemaphoreType.DMA,
            pltpu.SemaphoreType.DMA,
            pltpu.SemaphoreType.DMA,
            pltpu.SemaphoreType.DMA,
            pltpu.SemaphoreType.DMA,
            pltpu.SemaphoreType.DMA,
        ],
        compiler_params=pltpu.CompilerParams(use_tc_tiling_on_sc=False),
    )
    def gather_kernel(idx_hbm, table_hbm, out_hbm, iv0, iv1, rv0, rv1,
                      si0, si1, sg0, sg1, so0, so1):
        wid = lax.axis_index("s") * info.num_cores + lax.axis_index("c")
        base = wid * b_per_w

        def idx_slice(i):
            return idx_hbm.at[pl.ds(base + i * CHUNK, CHUNK)]

        def out_slice(i):
            return out_hbm.at[pl.ds(base + i * CHUNK, CHUNK)]

        pltpu.async_copy(idx_slice(0), iv0, si0)
        pltpu.async_copy(idx_slice(1), iv1, si1)
        pltpu.make_async_copy(idx_slice(0), iv0, si0).wait()
        pltpu.async_copy(table_hbm.at[iv0], rv0, sg0)

        def body(j, carry):
            i = 2 * j
            pltpu.make_async_copy(table_hbm.at[iv0], rv0, sg0).wait()
            pltpu.make_async_copy(idx_slice(i + 1), iv1, si1).wait()

            @pl.when(j > 0)
            def _():
                pltpu.make_async_copy(rv1, out_slice(i - 1), so1).wait()

            pltpu.async_copy(table_hbm.at[iv1], rv1, sg1)
            pltpu.async_copy(rv0, out_slice(i), so0)
            i2 = lax.rem(i + 2, n_chunks)
            i3 = lax.rem(i + 3, n_chunks)
            pltpu.async_copy(idx_slice(i2), iv0, si0)
            pltpu.make_async_copy(table_hbm.at[iv1], rv1, sg1).wait()
            pltpu.async_copy(rv1, out_slice(i + 1), so1)
            pltpu.async_copy(idx_slice(i3), iv1, si1)
            pltpu.make_async_copy(rv0, out_slice(i), so0).wait()
            pltpu.make_async_copy(idx_slice(i2), iv0, si0).wait()

            @pl.when(j < pairs - 1)
            def _():
                pltpu.async_copy(table_hbm.at[iv0], rv0, sg0)

            return carry

        lax.fori_loop(0, pairs, body, 0)
        pltpu.make_async_copy(rv1, out_slice(n_chunks - 1), so1).wait()
        pltpu.make_async_copy(idx_slice(1), iv1, si1).wait()

    return gather_kernel


def kernel(radius, table):
    B0, H = radius.shape
    V, D = table.shape
    flat_idx = radius.reshape(-1).astype(jnp.int32)
    out = _build(B0 * H, V, D)(flat_idx, table)
    return out.reshape(B0, H, D)
